# baseline (speedup 1.0000x reference)
import jax
import jax.numpy as jnp
from jax import lax
from jax.experimental import pallas as pl
from jax.experimental.pallas import tpu as pltpu
from jax.experimental.pallas import tpu_sc as plsc

B = 16384
D = 32
OUT_D = 3 * D + 1

NC = 2
NS = 16
NW = NC * NS
BPW = B // NW
CHUNK = 128
NCH = BPW // CHUNK
L = 16


def _body(cid_h, memi_h, zipi_h, age_h, scale_h,
          id_tab, mem_tab, zip_tab, out_h,
          cid_v, memi_v, zipi_v, age_v, scale_v,
          rows_id, rows_mem, rows_zip, agecol_v, sem):
    c = lax.axis_index("c")
    s = lax.axis_index("s")
    wid = s * NC + c
    cbase = wid * NCH
    base = wid * BPW

    pltpu.sync_copy(cid_h.at[pl.ds(cbase, NCH)], cid_v)
    pltpu.sync_copy(memi_h.at[pl.ds(cbase, NCH)], memi_v)
    pltpu.sync_copy(zipi_h.at[pl.ds(cbase, NCH)], zipi_v)
    pltpu.sync_copy(age_h.at[pl.ds(cbase, NCH)], age_v)
    pltpu.sync_copy(scale_h, scale_v)

    copies = []
    for j in range(NCH):
        copies.append(pltpu.async_copy(
            id_tab.at[cid_v.at[j]], rows_id.at[pl.ds(j * CHUNK, CHUNK)], sem))
        copies.append(pltpu.async_copy(
            mem_tab.at[memi_v.at[j]], rows_mem.at[pl.ds(j * CHUNK, CHUNK)], sem))
        copies.append(pltpu.async_copy(
            zip_tab.at[zipi_v.at[j]], rows_zip.at[pl.ds(j * CHUNK, CHUNK)], sem))

    mean = scale_v[pl.ds(0, L)]
    inv = scale_v[pl.ds(L, L)]
    zeros = jnp.zeros((L,), jnp.int32)
    lane = lax.iota(jnp.int32, L)
    for j in range(NCH):
        for k in range(CHUNK // L):
            a = age_v[j, pl.ds(k * L, L)]
            rowi = lane + (j * CHUNK + k * L)
            plsc.store_scatter(agecol_v, [rowi, zeros], (a - mean) * inv)

    for cp in copies:
        cp.wait()

    pltpu.sync_copy(rows_id, out_h.at[pl.ds(base, BPW), pl.ds(0, D)])
    pltpu.sync_copy(rows_mem, out_h.at[pl.ds(base, BPW), pl.ds(D, D)])
    pltpu.sync_copy(rows_zip, out_h.at[pl.ds(base, BPW), pl.ds(2 * D, D)])
    pltpu.sync_copy(agecol_v, out_h.at[pl.ds(base, BPW), pl.ds(3 * D, 1)])


@jax.jit
def _impl(cid2, memi2, zipi2, age2, scale, id_table, membership_table, zip_table):
    mesh = plsc.VectorSubcoreMesh(core_axis_name="c", subcore_axis_name="s")
    return pl.kernel(
        _body,
        out_type=jax.ShapeDtypeStruct((B, OUT_D), jnp.float32),
        mesh=mesh,
        compiler_params=pltpu.CompilerParams(
            use_tc_tiling_on_sc=False, needs_layout_passes=False),
        scratch_types=[
            pltpu.VMEM((NCH, CHUNK), jnp.int32),
            pltpu.VMEM((NCH, CHUNK), jnp.int32),
            pltpu.VMEM((NCH, CHUNK), jnp.int32),
            pltpu.VMEM((NCH, CHUNK), jnp.float32),
            pltpu.VMEM((2 * L,), jnp.float32),
            pltpu.VMEM((BPW, D), jnp.float32),
            pltpu.VMEM((BPW, D), jnp.float32),
            pltpu.VMEM((BPW, D), jnp.float32),
            pltpu.VMEM((BPW, 1), jnp.float32),
            pltpu.SemaphoreType.DMA,
        ],
    )(cid2, memi2, zipi2, age2, scale, id_table, membership_table, zip_table)


def kernel(customer_id, club_member_status, postal_code, age,
           id_table, membership_table, zip_table, age_mean, age_var):
    inv_std = lax.rsqrt(age_var.astype(jnp.float32))
    scale = jnp.concatenate([
        jnp.full((L,), age_mean, jnp.float32),
        jnp.full((L,), inv_std, jnp.float32),
    ])
    cid2 = customer_id.reshape(NW * NCH, CHUNK)
    memi2 = club_member_status.reshape(NW * NCH, CHUNK)
    zipi2 = postal_code.reshape(NW * NCH, CHUNK)
    age2 = age.reshape(NW * NCH, CHUNK)
    return _impl(cid2, memi2, zipi2, age2, scale,
                 id_table, membership_table, zip_table)
